# baseline (speedup 1.0000x reference)
import functools

import jax
import jax.numpy as jnp
import numpy as np
from jax import lax
from jax.experimental import pallas as pl
from jax.experimental.pallas import tpu as pltpu
from jax.experimental.pallas import tpu_sc as plsc

POOL = 16384
DFF = 3072
DFW = DFF // 2
DM = 768
NTOK = 8192
K = 8
NC, NS, LANES = 2, 16, 16
NW = NC * NS
CH = 4
TPC = NTOK // CH
TPW = TPC // NW
GRP = TPW // LANES
OUT_T = 8
NBUF = 4
UNROLL = 2

_f32 = jnp.float32
_bf16 = jnp.bfloat16
_i32 = jnp.int32


PBLK = 512


def _pack_body(x_ref, o_ref):
    x = x_ref[...]
    lo = pltpu.bitcast(x[:, :DFW], _i32)
    hi = pltpu.bitcast(x[:, DFW:], _i32)

    def rne16(b):
        return lax.shift_right_logical(
            b + 0x7FFF + (lax.shift_right_logical(b, 16) & 1), 16)

    o_ref[...] = rne16(lo) | lax.shift_left(rne16(hi), 16)


def _tc_pack(fp):
    return pl.pallas_call(
        _pack_body,
        grid=(POOL // PBLK,),
        in_specs=[pl.BlockSpec((PBLK, DFF), lambda i: (i, 0))],
        out_specs=pl.BlockSpec((PBLK, DFW), lambda i: (i, 0)),
        out_shape=jax.ShapeDtypeStruct((POOL, DFW), _i32),
    )(fp)


def _take16(v, idx):
    return lax.gather(
        v, idx[:, None],
        lax.GatherDimensionNumbers(offset_dims=(), collapsed_slice_dims=(0,),
                                   start_index_map=(0,)),
        (1,), mode=lax.GatherScatterMode.PROMISE_IN_BOUNDS)


def _sc_body(chunk, fp_hbm, idx_hbm, w_hbm, out_hbm,
             idx_v, w_v, rows_v, out_v, gsems, osems):
    wid = lax.axis_index("s") * NC + lax.axis_index("c")
    base = wid * TPW
    gbase = chunk * TPC + base

    pltpu.sync_copy(idx_hbm.at[pl.ds(gbase * K, TPW * K)], idx_v)
    pltpu.sync_copy(w_hbm.at[pl.ds(gbase * K, TPW * K)], w_v)

    lane = lax.iota(_i32, LANES)
    xor_idx = [lane ^ d for d in (1, 2, 4)]

    def softmax_step(g, carry):
        tw = w_v[pl.ds(g * LANES, LANES)]
        m = tw
        for x in xor_idx:
            m = jnp.maximum(m, _take16(m, x))
        e = jnp.exp(tw - m)
        s = e
        for x in xor_idx:
            s = s + _take16(s, x)
        w_v[pl.ds(g * LANES, LANES)] = e / s
        return carry
    lax.fori_loop(0, TPW * K // LANES, softmax_step, 0)

    def g_copy(t, b):
        return pltpu.make_async_copy(
            fp_hbm.at[idx_v.at[pl.ds(t * K, K)]], rows_v.at[b], gsems[b])

    def o_copy(row, ob):
        return pltpu.make_async_copy(
            out_v.at[ob], out_hbm.at[pl.ds(row, OUT_T)], osems[ob])

    for b in range(NBUF - 1):
        g_copy(b, b).start()

    def super_body(i, carry):
        for j in range(16):
            t = i * 16 + j
            b = j % NBUF
            ob = j // 8
            if j == 0:
                @pl.when(i > 0)
                def _w0():
                    o_copy(base + (i - 1) * 16, 0).wait()
            if j == 8:
                @pl.when(i > 0)
                def _w1():
                    o_copy(base + (i - 1) * 16 + OUT_T, 1).wait()

            @pl.when(t + NBUF - 1 < TPW)
            def _g():
                g_copy(t + NBUF - 1, (j + NBUF - 1) % NBUF).start()
            g_copy(t, b).wait()

            wchunk = w_v[pl.ds((i * 16 + j) // 2 * LANES, LANES)]
            spl = [jnp.broadcast_to(wchunk[(j % 2) * K + k], (LANES,))
                   for k in range(K)]
            splb = [plsc.pack(s, s, format=plsc.PackFormat.INTERLEAVED)
                    for s in spl]

            def chunk_body(c, carry2, _b=b, _ob=ob, _jj=j % 8, _splb=splb):
                for u in range(UNROLL):
                    off = (c * UNROLL + u) * LANES
                    pk = plsc.bitcast(rows_v[_b, 0, pl.ds(off, LANES)], _bf16)
                    acc = _splb[0] * pk
                    for k in range(1, K):
                        pk = plsc.bitcast(
                            rows_v[_b, k, pl.ds(off, LANES)], _bf16)
                        acc = acc + _splb[k] * pk
                    out_v[_ob, _jj, pl.ds(off, LANES)] = plsc.bitcast(
                        acc, _i32)
                return carry2
            lax.fori_loop(0, DFW // (LANES * UNROLL), chunk_body, 0)

            if j == 7:
                o_copy(base + i * 16, 0).start()
            if j == 15:
                o_copy(base + i * 16 + OUT_T, 1).start()
        return carry

    lax.fori_loop(0, GRP, super_body, 0)

    o_copy(base + (GRP - 1) * 16, 0).wait()
    o_copy(base + (GRP - 1) * 16 + OUT_T, 1).wait()


_sc_combine = [
    functools.partial(
        pl.kernel,
        out_type=jax.ShapeDtypeStruct((TPC, DFW), _i32),
        mesh=plsc.VectorSubcoreMesh(
            core_axis_name="c", subcore_axis_name="s",
            num_cores=NC, num_subcores=NS),
        compiler_params=pltpu.CompilerParams(needs_layout_passes=False),
        scratch_types=[
            pltpu.VMEM((TPW * K,), _i32),
            pltpu.VMEM((TPW * K,), _f32),
            pltpu.VMEM((NBUF, K, DFW), _i32),
            pltpu.VMEM((2, OUT_T, DFW), _i32),
            [pltpu.SemaphoreType.DMA] * NBUF,
            [pltpu.SemaphoreType.DMA] * 2,
        ],
    )(functools.partial(_sc_body, c))
    for c in range(CH)
]


TBLK = 512


def _tc_body(x_ref, w_ref, b_ref, o_ref):
    xw = x_ref[...]
    lo = pltpu.bitcast(lax.shift_left(xw, 16), _f32)
    hi = pltpu.bitcast(xw & jnp.int32(-65536), _f32)
    w = w_ref[...]

    def gelu(x):
        return 0.5 * x * (1.0 + lax.erf(x * (2.0 ** -0.5)))

    nt = (((1,), (1,)), ((), ()))
    o_ref[...] = (
        lax.dot_general(gelu(lo).astype(_bf16), w[:, :DFW].astype(_bf16),
                        nt, preferred_element_type=_f32)
        + lax.dot_general(gelu(hi).astype(_bf16), w[:, DFW:].astype(_bf16),
                          nt, preferred_element_type=_f32)
        + b_ref[...])


def _tc_gelu_matmul(x, w2, b2, chunk, prev=None):
    blk0 = chunk * (TPC // TBLK)
    args = [x, w2, b2]
    in_specs = [
        pl.BlockSpec((TBLK, DFW), lambda i: (i, 0)),
        pl.BlockSpec((DM, DFF), lambda i: (0, 0)),
        pl.BlockSpec((1, DM), lambda i: (0, 0)),
    ]
    kwargs = {}
    if prev is not None:
        args.append(prev)
        in_specs.append(pl.BlockSpec((TBLK, DM), lambda i: (i, 0)))
        kwargs["input_output_aliases"] = {3: 0}
    return pl.pallas_call(
        lambda *refs: _tc_body(*refs[:3], refs[-1]),
        grid=(TPC // TBLK,),
        in_specs=in_specs,
        out_specs=pl.BlockSpec((TBLK, DM), lambda i, _b=blk0: (i + _b, 0)),
        out_shape=jax.ShapeDtypeStruct((NTOK, DM), _f32),
        **kwargs,
    )(*args)


def kernel(selected_indices, pattern_weights, firing_patterns, W2_w, W2_b):
    B, S, _ = selected_indices.shape
    idx = selected_indices.reshape(NTOK * K).astype(_i32)
    wflat = pattern_weights.reshape(NTOK * K)
    fpi = _tc_pack(firing_patterns)
    b2 = W2_b.reshape(1, DM)
    combined = [_sc_combine[c](fpi, idx, wflat) for c in range(CH)]
    out = None
    for c in range(CH):
        out = _tc_gelu_matmul(combined[c], W2_w, b2, c, prev=out)
    return out.reshape(B, S, DM)


# speedup vs baseline: 1.0369x; 1.0369x over previous
import functools

import jax
import jax.numpy as jnp
import numpy as np
from jax import lax
from jax.experimental import pallas as pl
from jax.experimental.pallas import tpu as pltpu
from jax.experimental.pallas import tpu_sc as plsc

POOL = 16384
DFF = 3072
DFW = DFF // 2
DM = 768
NTOK = 8192
K = 8
NC, NS, LANES = 2, 16, 16
NW = NC * NS
CH = 2
TPC = NTOK // CH
TPW = TPC // NW
GRP = TPW // LANES
OUT_T = 8
NBUF = 4
UNROLL = 2

_f32 = jnp.float32
_bf16 = jnp.bfloat16
_i32 = jnp.int32


PBLK = 1024


def _pack_body(x_ref, o_ref):
    x = x_ref[...]
    lo = pltpu.bitcast(x[:, :DFW], _i32)
    hi = pltpu.bitcast(x[:, DFW:], _i32)

    def rne16(b):
        return lax.shift_right_logical(
            b + 0x7FFF + (lax.shift_right_logical(b, 16) & 1), 16)

    o_ref[...] = rne16(lo) | lax.shift_left(rne16(hi), 16)


def _tc_pack(fp):
    return pl.pallas_call(
        _pack_body,
        grid=(POOL // PBLK,),
        in_specs=[pl.BlockSpec((PBLK, DFF), lambda i: (i, 0))],
        out_specs=pl.BlockSpec((PBLK, DFW), lambda i: (i, 0)),
        out_shape=jax.ShapeDtypeStruct((POOL, DFW), _i32),
    )(fp)


def _take16(v, idx):
    return lax.gather(
        v, idx[:, None],
        lax.GatherDimensionNumbers(offset_dims=(), collapsed_slice_dims=(0,),
                                   start_index_map=(0,)),
        (1,), mode=lax.GatherScatterMode.PROMISE_IN_BOUNDS)


def _sc_body(chunk, fp_hbm, idx_hbm, w_hbm, out_hbm,
             idx_v, w_v, rows_v, out_v, gsems, osems):
    wid = lax.axis_index("s") * NC + lax.axis_index("c")
    base = wid * TPW
    gbase = chunk * TPC + base

    pltpu.sync_copy(idx_hbm.at[pl.ds(gbase * K, TPW * K)], idx_v)
    pltpu.sync_copy(w_hbm.at[pl.ds(gbase * K, TPW * K)], w_v)

    lane = lax.iota(_i32, LANES)
    xor_idx = [lane ^ d for d in (1, 2, 4)]

    def softmax_step(g, carry):
        tw = w_v[pl.ds(g * LANES, LANES)]
        m = tw
        for x in xor_idx:
            m = jnp.maximum(m, _take16(m, x))
        e = jnp.exp(tw - m)
        s = e
        for x in xor_idx:
            s = s + _take16(s, x)
        w_v[pl.ds(g * LANES, LANES)] = e / s
        return carry
    lax.fori_loop(0, TPW * K // LANES, softmax_step, 0)

    def g_copy(t, b):
        return pltpu.make_async_copy(
            fp_hbm.at[idx_v.at[pl.ds(t * K, K)]], rows_v.at[b], gsems[b])

    def o_copy(row, ob):
        return pltpu.make_async_copy(
            out_v.at[ob], out_hbm.at[pl.ds(row, OUT_T)], osems[ob])

    for b in range(NBUF - 1):
        g_copy(b, b).start()

    def super_body(i, carry):
        for j in range(16):
            t = i * 16 + j
            b = j % NBUF
            ob = j // 8
            if j == 0:
                @pl.when(i > 0)
                def _w0():
                    o_copy(base + (i - 1) * 16, 0).wait()
            if j == 8:
                @pl.when(i > 0)
                def _w1():
                    o_copy(base + (i - 1) * 16 + OUT_T, 1).wait()

            @pl.when(t + NBUF - 1 < TPW)
            def _g():
                g_copy(t + NBUF - 1, (j + NBUF - 1) % NBUF).start()
            g_copy(t, b).wait()

            wchunk = w_v[pl.ds((i * 16 + j) // 2 * LANES, LANES)]
            spl = [jnp.broadcast_to(wchunk[(j % 2) * K + k], (LANES,))
                   for k in range(K)]
            splb = [plsc.pack(s, s, format=plsc.PackFormat.INTERLEAVED)
                    for s in spl]

            def chunk_body(c, carry2, _b=b, _ob=ob, _jj=j % 8, _splb=splb):
                for u in range(UNROLL):
                    off = (c * UNROLL + u) * LANES
                    pk = plsc.bitcast(rows_v[_b, 0, pl.ds(off, LANES)], _bf16)
                    acc = _splb[0] * pk
                    for k in range(1, K):
                        pk = plsc.bitcast(
                            rows_v[_b, k, pl.ds(off, LANES)], _bf16)
                        acc = acc + _splb[k] * pk
                    out_v[_ob, _jj, pl.ds(off, LANES)] = plsc.bitcast(
                        acc, _i32)
                return carry2
            lax.fori_loop(0, DFW // (LANES * UNROLL), chunk_body, 0)

            if j == 7:
                o_copy(base + i * 16, 0).start()
            if j == 15:
                o_copy(base + i * 16 + OUT_T, 1).start()
        return carry

    lax.fori_loop(0, GRP, super_body, 0)

    o_copy(base + (GRP - 1) * 16, 0).wait()
    o_copy(base + (GRP - 1) * 16 + OUT_T, 1).wait()


_sc_combine = [
    functools.partial(
        pl.kernel,
        out_type=jax.ShapeDtypeStruct((TPC, DFW), _i32),
        mesh=plsc.VectorSubcoreMesh(
            core_axis_name="c", subcore_axis_name="s",
            num_cores=NC, num_subcores=NS),
        compiler_params=pltpu.CompilerParams(needs_layout_passes=False),
        scratch_types=[
            pltpu.VMEM((TPW * K,), _i32),
            pltpu.VMEM((TPW * K,), _f32),
            pltpu.VMEM((NBUF, K, DFW), _i32),
            pltpu.VMEM((2, OUT_T, DFW), _i32),
            [pltpu.SemaphoreType.DMA] * NBUF,
            [pltpu.SemaphoreType.DMA] * 2,
        ],
    )(functools.partial(_sc_body, c))
    for c in range(CH)
]


TBLK = 1024


def _tc_body(x_ref, w_ref, b_ref, o_ref):
    xw = x_ref[...]
    lo = pltpu.bitcast(lax.shift_left(xw, 16), _f32)
    hi = pltpu.bitcast(xw & jnp.int32(-65536), _f32)
    w = w_ref[...]

    def gelu(x):
        return 0.5 * x * (1.0 + lax.erf(x * (2.0 ** -0.5)))

    nt = (((1,), (1,)), ((), ()))
    o_ref[...] = (
        lax.dot_general(gelu(lo).astype(_bf16), w[:, :DFW].astype(_bf16),
                        nt, preferred_element_type=_f32)
        + lax.dot_general(gelu(hi).astype(_bf16), w[:, DFW:].astype(_bf16),
                          nt, preferred_element_type=_f32)
        + b_ref[...])


def _tc_gelu_matmul(x, w2, b2, chunk, prev=None):
    blk0 = chunk * (TPC // TBLK)
    args = [x, w2, b2]
    in_specs = [
        pl.BlockSpec((TBLK, DFW), lambda i: (i, 0)),
        pl.BlockSpec((DM, DFF), lambda i: (0, 0)),
        pl.BlockSpec((1, DM), lambda i: (0, 0)),
    ]
    kwargs = {}
    if prev is not None:
        args.append(prev)
        in_specs.append(pl.BlockSpec((TBLK, DM), lambda i: (i, 0)))
        kwargs["input_output_aliases"] = {3: 0}
    return pl.pallas_call(
        lambda *refs: _tc_body(*refs[:3], refs[-1]),
        grid=(TPC // TBLK,),
        in_specs=in_specs,
        out_specs=pl.BlockSpec((TBLK, DM), lambda i, _b=blk0: (i + _b, 0)),
        out_shape=jax.ShapeDtypeStruct((NTOK, DM), _f32),
        **kwargs,
    )(*args)


def kernel(selected_indices, pattern_weights, firing_patterns, W2_w, W2_b):
    B, S, _ = selected_indices.shape
    idx = selected_indices.reshape(NTOK * K).astype(_i32)
    wflat = pattern_weights.reshape(NTOK * K)
    fpi = _tc_pack(firing_patterns)
    b2 = W2_b.reshape(1, DM)
    combined = [_sc_combine[c](fpi, idx, wflat) for c in range(CH)]
    out = None
    for c in range(CH):
        out = _tc_gelu_matmul(combined[c], W2_w, b2, c, prev=out)
    return out.reshape(B, S, DM)


# speedup vs baseline: 1.0446x; 1.0074x over previous
import functools

import jax
import jax.numpy as jnp
import numpy as np
from jax import lax
from jax.experimental import pallas as pl
from jax.experimental.pallas import tpu as pltpu
from jax.experimental.pallas import tpu_sc as plsc

POOL = 16384
DFF = 3072
DFW = DFF // 2
DM = 768
NTOK = 8192
K = 8
NC, NS, LANES = 2, 16, 16
NW = NC * NS
CH = 2
TPC = NTOK // CH
TPW = TPC // NW
GRP = TPW // LANES
OUT_T = 8
NBUF = 4
UNROLL = 4

_f32 = jnp.float32
_bf16 = jnp.bfloat16
_i32 = jnp.int32


PBLK = 1024


def _pack_body(x_ref, o_ref):
    x = x_ref[...]
    lo = pltpu.bitcast(x[:, :DFW], _i32)
    hi = pltpu.bitcast(x[:, DFW:], _i32)

    def rne16(b):
        return lax.shift_right_logical(
            b + 0x7FFF + (lax.shift_right_logical(b, 16) & 1), 16)

    o_ref[...] = rne16(lo) | lax.shift_left(rne16(hi), 16)


def _tc_pack(fp):
    return pl.pallas_call(
        _pack_body,
        grid=(POOL // PBLK,),
        in_specs=[pl.BlockSpec((PBLK, DFF), lambda i: (i, 0))],
        out_specs=pl.BlockSpec((PBLK, DFW), lambda i: (i, 0)),
        out_shape=jax.ShapeDtypeStruct((POOL, DFW), _i32),
    )(fp)


def _take16(v, idx):
    return lax.gather(
        v, idx[:, None],
        lax.GatherDimensionNumbers(offset_dims=(), collapsed_slice_dims=(0,),
                                   start_index_map=(0,)),
        (1,), mode=lax.GatherScatterMode.PROMISE_IN_BOUNDS)


def _sc_body(chunk, fp_hbm, idx_hbm, w_hbm, out_hbm,
             idx_v, w_v, rows_v, out_v, gsems, osems):
    wid = lax.axis_index("s") * NC + lax.axis_index("c")
    base = wid * TPW
    gbase = chunk * TPC + base

    pltpu.sync_copy(idx_hbm.at[pl.ds(gbase * K, TPW * K)], idx_v)
    pltpu.sync_copy(w_hbm.at[pl.ds(gbase * K, TPW * K)], w_v)

    lane = lax.iota(_i32, LANES)
    xor_idx = [lane ^ d for d in (1, 2, 4)]

    def softmax_step(g, carry):
        tw = w_v[pl.ds(g * LANES, LANES)]
        m = tw
        for x in xor_idx:
            m = jnp.maximum(m, _take16(m, x))
        e = jnp.exp(tw - m)
        s = e
        for x in xor_idx:
            s = s + _take16(s, x)
        w_v[pl.ds(g * LANES, LANES)] = e / s
        return carry
    lax.fori_loop(0, TPW * K // LANES, softmax_step, 0)

    def g_copy(t, b):
        return pltpu.make_async_copy(
            fp_hbm.at[idx_v.at[pl.ds(t * K, K)]], rows_v.at[b], gsems[b])

    def o_copy(row, ob):
        return pltpu.make_async_copy(
            out_v.at[ob], out_hbm.at[pl.ds(row, OUT_T)], osems[ob])

    for b in range(NBUF - 1):
        g_copy(b, b).start()

    def super_body(i, carry):
        for j in range(16):
            t = i * 16 + j
            b = j % NBUF
            ob = j // 8
            if j == 0:
                @pl.when(i > 0)
                def _w0():
                    o_copy(base + (i - 1) * 16, 0).wait()
            if j == 8:
                @pl.when(i > 0)
                def _w1():
                    o_copy(base + (i - 1) * 16 + OUT_T, 1).wait()

            @pl.when(t + NBUF - 1 < TPW)
            def _g():
                g_copy(t + NBUF - 1, (j + NBUF - 1) % NBUF).start()
            g_copy(t, b).wait()

            wchunk = w_v[pl.ds((i * 16 + j) // 2 * LANES, LANES)]
            spl = [jnp.broadcast_to(wchunk[(j % 2) * K + k], (LANES,))
                   for k in range(K)]
            splb = [plsc.pack(s, s, format=plsc.PackFormat.INTERLEAVED)
                    for s in spl]

            def chunk_body(c, carry2, _b=b, _ob=ob, _jj=j % 8, _splb=splb):
                for u in range(UNROLL):
                    off = (c * UNROLL + u) * LANES
                    pk = plsc.bitcast(rows_v[_b, 0, pl.ds(off, LANES)], _bf16)
                    acc = _splb[0] * pk
                    for k in range(1, K):
                        pk = plsc.bitcast(
                            rows_v[_b, k, pl.ds(off, LANES)], _bf16)
                        acc = acc + _splb[k] * pk
                    out_v[_ob, _jj, pl.ds(off, LANES)] = plsc.bitcast(
                        acc, _i32)
                return carry2
            lax.fori_loop(0, DFW // (LANES * UNROLL), chunk_body, 0)

            if j == 7:
                o_copy(base + i * 16, 0).start()
            if j == 15:
                o_copy(base + i * 16 + OUT_T, 1).start()
        return carry

    lax.fori_loop(0, GRP, super_body, 0)

    o_copy(base + (GRP - 1) * 16, 0).wait()
    o_copy(base + (GRP - 1) * 16 + OUT_T, 1).wait()


_sc_combine = [
    functools.partial(
        pl.kernel,
        out_type=jax.ShapeDtypeStruct((TPC, DFW), _i32),
        mesh=plsc.VectorSubcoreMesh(
            core_axis_name="c", subcore_axis_name="s",
            num_cores=NC, num_subcores=NS),
        compiler_params=pltpu.CompilerParams(needs_layout_passes=False),
        scratch_types=[
            pltpu.VMEM((TPW * K,), _i32),
            pltpu.VMEM((TPW * K,), _f32),
            pltpu.VMEM((NBUF, K, DFW), _i32),
            pltpu.VMEM((2, OUT_T, DFW), _i32),
            [pltpu.SemaphoreType.DMA] * NBUF,
            [pltpu.SemaphoreType.DMA] * 2,
        ],
    )(functools.partial(_sc_body, c))
    for c in range(CH)
]


TBLK = 1024


def _tc_body(x_ref, w_ref, b_ref, o_ref):
    xw = x_ref[...]
    lo = pltpu.bitcast(lax.shift_left(xw, 16), _f32)
    hi = pltpu.bitcast(xw & jnp.int32(-65536), _f32)
    w = w_ref[...]

    def gelu(x):
        return 0.5 * x * (1.0 + lax.erf(x * (2.0 ** -0.5)))

    nt = (((1,), (1,)), ((), ()))
    o_ref[...] = (
        lax.dot_general(gelu(lo).astype(_bf16), w[:, :DFW].astype(_bf16),
                        nt, preferred_element_type=_f32)
        + lax.dot_general(gelu(hi).astype(_bf16), w[:, DFW:].astype(_bf16),
                          nt, preferred_element_type=_f32)
        + b_ref[...])


def _tc_gelu_matmul(x, w2, b2, chunk, prev=None):
    blk0 = chunk * (TPC // TBLK)
    args = [x, w2, b2]
    in_specs = [
        pl.BlockSpec((TBLK, DFW), lambda i: (i, 0)),
        pl.BlockSpec((DM, DFF), lambda i: (0, 0)),
        pl.BlockSpec((1, DM), lambda i: (0, 0)),
    ]
    kwargs = {}
    if prev is not None:
        args.append(prev)
        in_specs.append(pl.BlockSpec((TBLK, DM), lambda i: (i, 0)))
        kwargs["input_output_aliases"] = {3: 0}
    return pl.pallas_call(
        lambda *refs: _tc_body(*refs[:3], refs[-1]),
        grid=(TPC // TBLK,),
        in_specs=in_specs,
        out_specs=pl.BlockSpec((TBLK, DM), lambda i, _b=blk0: (i + _b, 0)),
        out_shape=jax.ShapeDtypeStruct((NTOK, DM), _f32),
        **kwargs,
    )(*args)


def kernel(selected_indices, pattern_weights, firing_patterns, W2_w, W2_b):
    B, S, _ = selected_indices.shape
    idx = selected_indices.reshape(NTOK * K).astype(_i32)
    wflat = pattern_weights.reshape(NTOK * K)
    fpi = _tc_pack(firing_patterns)
    b2 = W2_b.reshape(1, DM)
    combined = [_sc_combine[c](fpi, idx, wflat) for c in range(CH)]
    out = None
    for c in range(CH):
        out = _tc_gelu_matmul(combined[c], W2_w, b2, c, prev=out)
    return out.reshape(B, S, DM)


# speedup vs baseline: 1.5312x; 1.4659x over previous
import functools

import jax
import jax.numpy as jnp
import numpy as np
from jax import lax
from jax.experimental import pallas as pl
from jax.experimental.pallas import tpu as pltpu
from jax.experimental.pallas import tpu_sc as plsc

POOL = 16384
DFF = 3072
DFW = DFF // 2
DM = 768
NTOK = 8192
K = 8
NC, NS, LANES = 2, 16, 16
NW = NC * NS
CH = 2
TPC = NTOK // CH
TPW = TPC // NW
GRP = TPW // LANES
OUT_T = 8
NBUF = 4
UNROLL = 8

_f32 = jnp.float32
_bf16 = jnp.bfloat16
_i32 = jnp.int32


PBLK = 1024


def _pack_body(x_ref, o_ref):
    x = x_ref[...]
    lo = pltpu.bitcast(x[:, :DFW], _i32)
    hi = pltpu.bitcast(x[:, DFW:], _i32)

    def rne16(b):
        return lax.shift_right_logical(
            b + 0x7FFF + (lax.shift_right_logical(b, 16) & 1), 16)

    o_ref[...] = rne16(lo) | lax.shift_left(rne16(hi), 16)


def _tc_pack(fp):
    return pl.pallas_call(
        _pack_body,
        grid=(POOL // PBLK,),
        in_specs=[pl.BlockSpec((PBLK, DFF), lambda i: (i, 0))],
        out_specs=pl.BlockSpec((PBLK, DFW), lambda i: (i, 0)),
        out_shape=jax.ShapeDtypeStruct((POOL, DFW), _i32),
    )(fp)


def _take16(v, idx):
    return lax.gather(
        v, idx[:, None],
        lax.GatherDimensionNumbers(offset_dims=(), collapsed_slice_dims=(0,),
                                   start_index_map=(0,)),
        (1,), mode=lax.GatherScatterMode.PROMISE_IN_BOUNDS)


def _sc_body(chunk, fp_hbm, idx_hbm, w_hbm, out_hbm,
             idx_v, w_v, rows_v, out_v, gsems, osems):
    wid = lax.axis_index("s") * NC + lax.axis_index("c")
    base = wid * TPW
    gbase = chunk * TPC + base

    pltpu.sync_copy(idx_hbm.at[pl.ds(gbase * K, TPW * K)], idx_v)
    pltpu.sync_copy(w_hbm.at[pl.ds(gbase * K, TPW * K)], w_v)

    lane = lax.iota(_i32, LANES)
    xor_idx = [lane ^ d for d in (1, 2, 4)]

    def softmax_step(g, carry):
        tw = w_v[pl.ds(g * LANES, LANES)]
        m = tw
        for x in xor_idx:
            m = jnp.maximum(m, _take16(m, x))
        e = jnp.exp(tw - m)
        s = e
        for x in xor_idx:
            s = s + _take16(s, x)
        w_v[pl.ds(g * LANES, LANES)] = e / s
        return carry
    lax.fori_loop(0, TPW * K // LANES, softmax_step, 0)

    def g_copy(t, b):
        return pltpu.make_async_copy(
            fp_hbm.at[idx_v.at[pl.ds(t * K, K)]], rows_v.at[b], gsems[b])

    def o_copy(row, ob):
        return pltpu.make_async_copy(
            out_v.at[ob], out_hbm.at[pl.ds(row, OUT_T)], osems[ob])

    for b in range(NBUF - 1):
        g_copy(b, b).start()

    def super_body(i, carry):
        for j in range(16):
            t = i * 16 + j
            b = j % NBUF
            ob = j // 8
            if j == 0:
                @pl.when(i > 0)
                def _w0():
                    o_copy(base + (i - 1) * 16, 0).wait()
            if j == 8:
                @pl.when(i > 0)
                def _w1():
                    o_copy(base + (i - 1) * 16 + OUT_T, 1).wait()

            @pl.when(t + NBUF - 1 < TPW)
            def _g():
                g_copy(t + NBUF - 1, (j + NBUF - 1) % NBUF).start()
            g_copy(t, b).wait()

            wchunk = w_v[pl.ds((i * 16 + j) // 2 * LANES, LANES)]
            spl = [jnp.broadcast_to(wchunk[(j % 2) * K + k], (LANES,))
                   for k in range(K)]
            splb = [plsc.pack(s, s, format=plsc.PackFormat.INTERLEAVED)
                    for s in spl]

            def chunk_body(c, carry2, _b=b, _ob=ob, _jj=j % 8, _splb=splb):
                for u in range(UNROLL):
                    off = (c * UNROLL + u) * LANES
                    pk = plsc.bitcast(rows_v[_b, 0, pl.ds(off, LANES)], _bf16)
                    acc = _splb[0] * pk
                    for k in range(1, K):
                        pk = plsc.bitcast(
                            rows_v[_b, k, pl.ds(off, LANES)], _bf16)
                        acc = acc + _splb[k] * pk
                    out_v[_ob, _jj, pl.ds(off, LANES)] = plsc.bitcast(
                        acc, _i32)
                return carry2
            lax.fori_loop(0, DFW // (LANES * UNROLL), chunk_body, 0)

            if j == 7:
                o_copy(base + i * 16, 0).start()
            if j == 15:
                o_copy(base + i * 16 + OUT_T, 1).start()
        return carry

    lax.fori_loop(0, GRP, super_body, 0)

    o_copy(base + (GRP - 1) * 16, 0).wait()
    o_copy(base + (GRP - 1) * 16 + OUT_T, 1).wait()


_sc_combine = [
    functools.partial(
        pl.kernel,
        out_type=jax.ShapeDtypeStruct((TPC, DFW), _i32),
        mesh=plsc.VectorSubcoreMesh(
            core_axis_name="c", subcore_axis_name="s",
            num_cores=NC, num_subcores=NS),
        compiler_params=pltpu.CompilerParams(needs_layout_passes=False),
        scratch_types=[
            pltpu.VMEM((TPW * K,), _i32),
            pltpu.VMEM((TPW * K,), _f32),
            pltpu.VMEM((NBUF, K, DFW), _i32),
            pltpu.VMEM((2, OUT_T, DFW), _i32),
            [pltpu.SemaphoreType.DMA] * NBUF,
            [pltpu.SemaphoreType.DMA] * 2,
        ],
    )(functools.partial(_sc_body, c))
    for c in range(CH)
]


TBLK = 1024


def _tc_body(x_ref, w_ref, b_ref, o_ref):
    xw = x_ref[...]
    lo = pltpu.bitcast(lax.shift_left(xw, 16), _f32)
    hi = pltpu.bitcast(xw & jnp.int32(-65536), _f32)
    w = w_ref[...]

    def gelu(x):
        return 0.5 * x * (1.0 + lax.erf(x * (2.0 ** -0.5)))

    nt = (((1,), (1,)), ((), ()))
    o_ref[...] = (
        lax.dot_general(gelu(lo).astype(_bf16), w[:, :DFW].astype(_bf16),
                        nt, preferred_element_type=_f32)
        + lax.dot_general(gelu(hi).astype(_bf16), w[:, DFW:].astype(_bf16),
                          nt, preferred_element_type=_f32)
        + b_ref[...])


def _tc_gelu_matmul(x, w2, b2, chunk, prev=None):
    blk0 = chunk * (TPC // TBLK)
    args = [x, w2, b2]
    in_specs = [
        pl.BlockSpec((TBLK, DFW), lambda i: (i, 0)),
        pl.BlockSpec((DM, DFF), lambda i: (0, 0)),
        pl.BlockSpec((1, DM), lambda i: (0, 0)),
    ]
    kwargs = {}
    if prev is not None:
        args.append(prev)
        in_specs.append(pl.BlockSpec((TBLK, DM), lambda i: (i, 0)))
        kwargs["input_output_aliases"] = {3: 0}
    return pl.pallas_call(
        lambda *refs: _tc_body(*refs[:3], refs[-1]),
        grid=(TPC // TBLK,),
        in_specs=in_specs,
        out_specs=pl.BlockSpec((TBLK, DM), lambda i, _b=blk0: (i + _b, 0)),
        out_shape=jax.ShapeDtypeStruct((NTOK, DM), _f32),
        **kwargs,
    )(*args)


def kernel(selected_indices, pattern_weights, firing_patterns, W2_w, W2_b):
    B, S, _ = selected_indices.shape
    idx = selected_indices.reshape(NTOK * K).astype(_i32)
    wflat = pattern_weights.reshape(NTOK * K)
    fpi = _tc_pack(firing_patterns)
    b2 = W2_b.reshape(1, DM)
    combined = [_sc_combine[c](fpi, idx, wflat) for c in range(CH)]
    out = None
    for c in range(CH):
        out = _tc_gelu_matmul(combined[c], W2_w, b2, c, prev=out)
    return out.reshape(B, S, DM)
